# baseline (speedup 1.0000x reference)
import jax
import jax.numpy as jnp
from jax import lax
from jax.experimental import pallas as pl
from jax.experimental.pallas import tpu as pltpu
from jax.experimental.pallas import tpu_sc as plsc

N_TOTAL = 8388608
NC = 2
NS = 16
L = 16
NW = NC * NS

N_SC = (N_TOTAL // 16) * 3
PER_W = N_SC // NW
CHUNK = 16384
N_CHUNKS = PER_W // CHUNK
STEPS = CHUNK // L

TC_LANES = 128
TC_ROWS_TOTAL = N_TOTAL // TC_LANES
TC_ROW0 = N_SC // TC_LANES
TC_BLOCK_ROWS = 2048
TC_GRID = (TC_ROWS_TOTAL - TC_ROW0) // TC_BLOCK_ROWS

_C0 = 1.99869362
_C1 = 0.72011905
_SIGN = -2147483648


def _sc_body(x_hbm, t_hbm, out_hbm, xbuf, tbuf, accbuf, sems):
    wid = lax.axis_index("s") * NC + lax.axis_index("c")
    base = wid * PER_W

    def start(g):
        b = g % 2
        off = base + g * CHUNK
        cx = pltpu.async_copy(x_hbm.at[pl.ds(off, CHUNK)], xbuf.at[b],
                              sems.at[b, 0])
        ct = pltpu.async_copy(t_hbm.at[pl.ds(off, CHUNK)], tbuf.at[b],
                              sems.at[b, 1])
        return cx, ct

    def term(x, t):
        xi = lax.bitcast_convert_type(x, jnp.int32)
        u = lax.bitcast_convert_type(xi ^ (t << 31), jnp.float32)
        alpha = jnp.where(t > 0, 0.8, 0.2).astype(jnp.float32)
        na = lax.bitcast_convert_type(xi | _SIGN, jnp.float32)
        w = jnp.exp(na)
        r = 1.0 / (1.0 + w)
        sig = jnp.where(u >= 0.0, r, w * r)
        s = w / (2.0 + w)
        s2 = s * s
        l1p = s * (_C0 + s2 * _C1)
        sp = jnp.maximum(u, 0.0) + l1p
        return (alpha * sp) * (sig * sig)

    inflight = {0: start(0)}
    acc = jnp.zeros((L,), jnp.float32)
    for g in range(N_CHUNKS):
        if g + 1 < N_CHUNKS:
            inflight[g + 1] = start(g + 1)
        cx, ct = inflight.pop(g)
        cx.wait()
        ct.wait()
        b = g % 2
        xb = xbuf.at[b]
        tb = tbuf.at[b]

        def step(i, acc, xb=xb, tb=tb):
            x = xb[pl.ds(i * L, L)]
            t = tb[pl.ds(i * L, L)]
            return acc + term(x, t)

        acc = lax.fori_loop(0, STEPS, step, acc, unroll=16)

    accbuf[...] = acc
    pltpu.sync_copy(accbuf, out_hbm.at[wid])


_sc_partials = pl.kernel(
    _sc_body,
    out_type=jax.ShapeDtypeStruct((NW, L), jnp.float32),
    mesh=plsc.VectorSubcoreMesh(core_axis_name="c", subcore_axis_name="s",
                                num_cores=NC, num_subcores=NS),
    scratch_types=[
        pltpu.VMEM((2, CHUNK), jnp.float32),
        pltpu.VMEM((2, CHUNK), jnp.int32),
        pltpu.VMEM((L,), jnp.float32),
        pltpu.SemaphoreType.DMA((2, 2)),
    ],
)


def _tc_body(x_ref, t_ref, o_ref):
    @pl.when(pl.program_id(0) == 0)
    def _():
        o_ref[...] = jnp.zeros_like(o_ref)

    x = x_ref[...]
    t = t_ref[...]
    xi = lax.bitcast_convert_type(x, jnp.int32)
    u = lax.bitcast_convert_type(xi ^ (t << 31), jnp.float32)
    alpha = jnp.where(t > 0, 0.8, 0.2).astype(jnp.float32)
    na = lax.bitcast_convert_type(xi | _SIGN, jnp.float32)
    w = jnp.exp(na)
    r = 1.0 / (1.0 + w)
    sig = jnp.where(u >= 0.0, r, w * r)
    s = w / (2.0 + w)
    s2 = s * s
    l1p = s * (_C0 + s2 * _C1)
    sp = jnp.maximum(u, 0.0) + l1p
    term = (alpha * sp) * (sig * sig)
    o_ref[...] += term.reshape(TC_BLOCK_ROWS // 8, 8, TC_LANES).sum(axis=0)


_tc_partial = pl.pallas_call(
    _tc_body,
    grid=(TC_GRID,),
    in_specs=[
        pl.BlockSpec((TC_BLOCK_ROWS, TC_LANES),
                     lambda i: (i + TC_ROW0 // TC_BLOCK_ROWS, 0)),
        pl.BlockSpec((TC_BLOCK_ROWS, TC_LANES),
                     lambda i: (i + TC_ROW0 // TC_BLOCK_ROWS, 0)),
    ],
    out_specs=pl.BlockSpec((8, TC_LANES), lambda i: (0, 0)),
    out_shape=jax.ShapeDtypeStruct((8, TC_LANES), jnp.float32),
)


def kernel(inputs, targets):
    sc = _sc_partials(inputs, targets)
    tc = _tc_partial(inputs.reshape(TC_ROWS_TOTAL, TC_LANES),
                     targets.reshape(TC_ROWS_TOTAL, TC_LANES))
    return (jnp.sum(sc) + jnp.sum(tc)) * (1.0 / N_TOTAL)


# speedup vs baseline: 1.0665x; 1.0665x over previous
import jax
import jax.numpy as jnp
from jax import lax
from jax.experimental import pallas as pl
from jax.experimental.pallas import tpu as pltpu
from jax.experimental.pallas import tpu_sc as plsc

N_TOTAL = 8388608
NC = 2
NS = 16
L = 16
NW = NC * NS

N_SC = N_TOTAL // 4
PER_W = N_SC // NW
CHUNK = 16384
N_CHUNKS = PER_W // CHUNK
STEPS = CHUNK // L

TC_LANES = 128
TC_ROWS_TOTAL = N_TOTAL // TC_LANES
TC_ROW0 = N_SC // TC_LANES
TC_BLOCK_ROWS = 4096
TC_GRID = (TC_ROWS_TOTAL - TC_ROW0) // TC_BLOCK_ROWS

_C0 = 1.99869362
_C1 = 0.72011905
_SIGN = -2147483648


def _sc_body(x_hbm, t_hbm, out_hbm, xbuf, tbuf, accbuf, sems):
    wid = lax.axis_index("s") * NC + lax.axis_index("c")
    base = wid * PER_W

    def start(g):
        b = g % 2
        off = base + g * CHUNK
        cx = pltpu.async_copy(x_hbm.at[pl.ds(off, CHUNK)], xbuf.at[b],
                              sems.at[b, 0])
        ct = pltpu.async_copy(t_hbm.at[pl.ds(off, CHUNK)], tbuf.at[b],
                              sems.at[b, 1])
        return cx, ct

    def term(x, t):
        xi = lax.bitcast_convert_type(x, jnp.int32)
        u = lax.bitcast_convert_type(xi ^ (t << 31), jnp.float32)
        alpha = jnp.where(t > 0, 0.8, 0.2).astype(jnp.float32)
        na = lax.bitcast_convert_type(xi | _SIGN, jnp.float32)
        w = jnp.exp(na)
        r = 1.0 / (1.0 + w)
        sig = jnp.where(u >= 0.0, r, w * r)
        s = w / (2.0 + w)
        s2 = s * s
        l1p = s * (_C0 + s2 * _C1)
        sp = jnp.maximum(u, 0.0) + l1p
        return (alpha * sp) * (sig * sig)

    inflight = {0: start(0)}
    acc = jnp.zeros((L,), jnp.float32)
    for g in range(N_CHUNKS):
        if g + 1 < N_CHUNKS:
            inflight[g + 1] = start(g + 1)
        cx, ct = inflight.pop(g)
        cx.wait()
        ct.wait()
        b = g % 2
        xb = xbuf.at[b]
        tb = tbuf.at[b]

        def step(i, acc, xb=xb, tb=tb):
            x = xb[pl.ds(i * L, L)]
            t = tb[pl.ds(i * L, L)]
            return acc + term(x, t)

        acc = lax.fori_loop(0, STEPS, step, acc, unroll=16)

    accbuf[...] = acc
    pltpu.sync_copy(accbuf, out_hbm.at[wid])


_sc_partials = pl.kernel(
    _sc_body,
    out_type=jax.ShapeDtypeStruct((NW, L), jnp.float32),
    mesh=plsc.VectorSubcoreMesh(core_axis_name="c", subcore_axis_name="s",
                                num_cores=NC, num_subcores=NS),
    scratch_types=[
        pltpu.VMEM((2, CHUNK), jnp.float32),
        pltpu.VMEM((2, CHUNK), jnp.int32),
        pltpu.VMEM((L,), jnp.float32),
        pltpu.SemaphoreType.DMA((2, 2)),
    ],
)


def _tc_body(x_ref, t_ref, o_ref):
    @pl.when(pl.program_id(0) == 0)
    def _():
        o_ref[...] = jnp.zeros_like(o_ref)

    x = x_ref[...]
    t = t_ref[...]
    xi = lax.bitcast_convert_type(x, jnp.int32)
    u = lax.bitcast_convert_type(xi ^ (t << 31), jnp.float32)
    alpha = jnp.where(t > 0, 0.8, 0.2).astype(jnp.float32)
    na = lax.bitcast_convert_type(xi | _SIGN, jnp.float32)
    w = jnp.exp(na)
    r = 1.0 / (1.0 + w)
    sig = jnp.where(u >= 0.0, r, w * r)
    s = w / (2.0 + w)
    s2 = s * s
    l1p = s * (_C0 + s2 * _C1)
    sp = jnp.maximum(u, 0.0) + l1p
    term = (alpha * sp) * (sig * sig)
    o_ref[...] += term.reshape(TC_BLOCK_ROWS // 8, 8, TC_LANES).sum(axis=0)


_tc_partial = pl.pallas_call(
    _tc_body,
    grid=(TC_GRID,),
    in_specs=[
        pl.BlockSpec((TC_BLOCK_ROWS, TC_LANES),
                     lambda i: (i + TC_ROW0 // TC_BLOCK_ROWS, 0)),
        pl.BlockSpec((TC_BLOCK_ROWS, TC_LANES),
                     lambda i: (i + TC_ROW0 // TC_BLOCK_ROWS, 0)),
    ],
    out_specs=pl.BlockSpec((8, TC_LANES), lambda i: (0, 0)),
    out_shape=jax.ShapeDtypeStruct((8, TC_LANES), jnp.float32),
)


def kernel(inputs, targets):
    sc = _sc_partials(inputs, targets)
    tc = _tc_partial(inputs.reshape(TC_ROWS_TOTAL, TC_LANES),
                     targets.reshape(TC_ROWS_TOTAL, TC_LANES))
    return (jnp.sum(sc) + jnp.sum(tc)) * (1.0 / N_TOTAL)


# speedup vs baseline: 1.0884x; 1.0206x over previous
import jax
import jax.numpy as jnp
from jax import lax
from jax.experimental import pallas as pl
from jax.experimental.pallas import tpu as pltpu
from jax.experimental.pallas import tpu_sc as plsc

N_TOTAL = 8388608
NC = 2
NS = 16
L = 16
NW = NC * NS

N_SC = N_TOTAL // 4
PER_W = N_SC // NW
CHUNK = 16384
N_CHUNKS = PER_W // CHUNK
STEPS = CHUNK // L

TC_LANES = 128
TC_ROWS_TOTAL = N_TOTAL // TC_LANES
TC_ROW0 = N_SC // TC_LANES
TC_BLOCK_ROWS = 8192
TC_GRID = (TC_ROWS_TOTAL - TC_ROW0) // TC_BLOCK_ROWS

_C0 = 1.99869362
_C1 = 0.72011905
_SIGN = -2147483648


def _sc_body(x_hbm, t_hbm, out_hbm, xbuf, tbuf, accbuf, sems):
    wid = lax.axis_index("s") * NC + lax.axis_index("c")
    base = wid * PER_W

    def start(g):
        b = g % 2
        off = base + g * CHUNK
        cx = pltpu.async_copy(x_hbm.at[pl.ds(off, CHUNK)], xbuf.at[b],
                              sems.at[b, 0])
        ct = pltpu.async_copy(t_hbm.at[pl.ds(off, CHUNK)], tbuf.at[b],
                              sems.at[b, 1])
        return cx, ct

    def term(x, t):
        xi = lax.bitcast_convert_type(x, jnp.int32)
        u = lax.bitcast_convert_type(xi ^ (t << 31), jnp.float32)
        alpha = jnp.where(t > 0, 0.8, 0.2).astype(jnp.float32)
        na = lax.bitcast_convert_type(xi | _SIGN, jnp.float32)
        w = jnp.exp(na)
        r = 1.0 / (1.0 + w)
        sig = jnp.where(u >= 0.0, r, w * r)
        s = w / (2.0 + w)
        s2 = s * s
        l1p = s * (_C0 + s2 * _C1)
        sp = jnp.maximum(u, 0.0) + l1p
        return (alpha * sp) * (sig * sig)

    inflight = {0: start(0)}
    acc = jnp.zeros((L,), jnp.float32)
    for g in range(N_CHUNKS):
        if g + 1 < N_CHUNKS:
            inflight[g + 1] = start(g + 1)
        cx, ct = inflight.pop(g)
        cx.wait()
        ct.wait()
        b = g % 2
        xb = xbuf.at[b]
        tb = tbuf.at[b]

        def step(i, acc, xb=xb, tb=tb):
            x = xb[pl.ds(i * L, L)]
            t = tb[pl.ds(i * L, L)]
            return acc + term(x, t)

        acc = lax.fori_loop(0, STEPS, step, acc, unroll=16)

    accbuf[...] = acc
    pltpu.sync_copy(accbuf, out_hbm.at[wid])


_sc_partials = pl.kernel(
    _sc_body,
    out_type=jax.ShapeDtypeStruct((NW, L), jnp.float32),
    mesh=plsc.VectorSubcoreMesh(core_axis_name="c", subcore_axis_name="s",
                                num_cores=NC, num_subcores=NS),
    scratch_types=[
        pltpu.VMEM((2, CHUNK), jnp.float32),
        pltpu.VMEM((2, CHUNK), jnp.int32),
        pltpu.VMEM((L,), jnp.float32),
        pltpu.SemaphoreType.DMA((2, 2)),
    ],
)


def _tc_body(x_ref, t_ref, o_ref):
    @pl.when(pl.program_id(0) == 0)
    def _():
        o_ref[...] = jnp.zeros_like(o_ref)

    x = x_ref[...]
    t = t_ref[...]
    xi = lax.bitcast_convert_type(x, jnp.int32)
    u = lax.bitcast_convert_type(xi ^ (t << 31), jnp.float32)
    alpha = jnp.where(t > 0, 0.8, 0.2).astype(jnp.float32)
    na = lax.bitcast_convert_type(xi | _SIGN, jnp.float32)
    w = jnp.exp(na)
    r = 1.0 / (1.0 + w)
    sig = jnp.where(u >= 0.0, r, w * r)
    s = w / (2.0 + w)
    s2 = s * s
    l1p = s * (_C0 + s2 * _C1)
    sp = jnp.maximum(u, 0.0) + l1p
    term = (alpha * sp) * (sig * sig)
    o_ref[...] += term.reshape(TC_BLOCK_ROWS // 8, 8, TC_LANES).sum(axis=0)


_tc_partial = pl.pallas_call(
    _tc_body,
    grid=(TC_GRID,),
    in_specs=[
        pl.BlockSpec((TC_BLOCK_ROWS, TC_LANES),
                     lambda i: (i + TC_ROW0 // TC_BLOCK_ROWS, 0)),
        pl.BlockSpec((TC_BLOCK_ROWS, TC_LANES),
                     lambda i: (i + TC_ROW0 // TC_BLOCK_ROWS, 0)),
    ],
    out_specs=pl.BlockSpec((8, TC_LANES), lambda i: (0, 0)),
    out_shape=jax.ShapeDtypeStruct((8, TC_LANES), jnp.float32),
)


def kernel(inputs, targets):
    sc = _sc_partials(inputs, targets)
    tc = _tc_partial(inputs.reshape(TC_ROWS_TOTAL, TC_LANES),
                     targets.reshape(TC_ROWS_TOTAL, TC_LANES))
    return (jnp.sum(sc) + jnp.sum(tc)) * (1.0 / N_TOTAL)
